# baseline (speedup 1.0000x reference)
import functools

import jax
import jax.numpy as jnp
from jax.experimental import pallas as pl

_K = 5
_BIG = 3.0e38
_BIGI = 1 << 30


def _row_norms_kernel(x_ref, out_ref):
    x = x_ref[...]
    out_ref[...] = jnp.sum(x * x, axis=1, keepdims=True)


def _knn_loss_kernel(x_row_ref, x_full_ref, sq_col_ref,
                     lab_row_ref, lab_col_ref, out_ref,
                     *, n_total, block_rows, num_blocks):
    i = pl.program_id(0)
    x = x_row_ref[...]
    xf = x_full_ref[...]

    sq_r = jnp.sum(x * x, axis=1, keepdims=True)
    sq_c = sq_col_ref[...]

    dot = jax.lax.dot_general(x, xf, (((1,), (1,)), ((), ())),
                              preferred_element_type=jnp.float32)
    d2 = jnp.maximum(sq_r + sq_c - 2.0 * dot, 0.0)
    dist = jnp.sqrt(d2)

    row_ids = i * block_rows + jax.lax.broadcasted_iota(
        jnp.int32, dist.shape, 0)
    col_ids = jax.lax.broadcasted_iota(jnp.int32, dist.shape, 1)
    dist = jnp.where(row_ids == col_ids, _BIG, dist)

    match = lab_row_ref[...] == lab_col_ref[...]
    mcol = 2 * col_ids + 1 - match.astype(jnp.int32)

    d0 = None
    sum_md = jnp.zeros_like(sq_r)
    sum_m = jnp.zeros_like(sq_r)
    sum_e = jnp.zeros_like(sq_r)
    dk = jnp.min(dist, axis=1, keepdims=True)
    for k in range(_K):
        first = jnp.min(jnp.where(dist == dk, mcol, _BIGI),
                        axis=1, keepdims=True)
        mk = (1 - (first & 1)).astype(jnp.float32)
        if d0 is None:
            d0 = dk
        sum_md += mk * dk
        sum_m += mk
        sum_e += jnp.exp(d0 - dk)
        if k < _K - 1:
            dist = jnp.where(mcol == first, _BIG, dist)
            dk = jnp.min(dist, axis=1, keepdims=True)

    lse = jnp.log(sum_e) - d0
    block_sum = jnp.sum(sum_md + sum_m * lse).reshape(1, 1)

    @pl.when(i == 0)
    def _init():
        out_ref[...] = jnp.zeros((1, 1), jnp.float32)

    out_ref[...] += block_sum

    @pl.when(i == num_blocks - 1)
    def _finish():
        out_ref[...] = out_ref[...] / n_total


def kernel(batch, labels):
    n, d = batch.shape
    block_rows = 256
    num_blocks = n // block_rows
    lab_row = labels.reshape(n, 1)
    lab_col = labels.reshape(1, n)

    sq = pl.pallas_call(
        _row_norms_kernel,
        grid=(num_blocks,),
        in_specs=[pl.BlockSpec((block_rows, d), lambda i: (i, 0))],
        out_specs=pl.BlockSpec((block_rows, 1), lambda i: (i, 0)),
        out_shape=jax.ShapeDtypeStruct((n, 1), jnp.float32),
    )(batch)
    sq_col = sq.reshape(1, n)

    body = functools.partial(_knn_loss_kernel, n_total=n,
                             block_rows=block_rows, num_blocks=num_blocks)
    out = pl.pallas_call(
        body,
        grid=(num_blocks,),
        in_specs=[
            pl.BlockSpec((block_rows, d), lambda i: (i, 0)),
            pl.BlockSpec((n, d), lambda i: (0, 0)),
            pl.BlockSpec((1, n), lambda i: (0, 0)),
            pl.BlockSpec((block_rows, 1), lambda i: (i, 0)),
            pl.BlockSpec((1, n), lambda i: (0, 0)),
        ],
        out_specs=pl.BlockSpec((1, 1), lambda i: (0, 0)),
        out_shape=jax.ShapeDtypeStruct((1, 1), jnp.float32),
    )(batch, batch, sq_col, lab_row, lab_col)
    return out[0, 0]


# speedup vs baseline: 1.2182x; 1.2182x over previous
import functools

import jax
import jax.numpy as jnp
from jax.experimental import pallas as pl

_K = 5
_BIG = 3.0e38
_BIGI = 1 << 30


def _row_norms_kernel(x_ref, out_ref):
    x = x_ref[...]
    out_ref[...] = jnp.sum(x * x, axis=1, keepdims=True)


def _knn_loss_kernel(x_row_ref, x_full_ref, sq_col_ref,
                     lab_row_ref, lab_col_ref, out_ref,
                     *, n_total, block_rows, num_blocks):
    i = pl.program_id(0)
    x = x_row_ref[...]
    xf = x_full_ref[...]

    sq_r = jnp.sum(x * x, axis=1, keepdims=True)
    sq_c = sq_col_ref[...]

    dot = jax.lax.dot_general(x, xf, (((1,), (1,)), ((), ())),
                              preferred_element_type=jnp.float32)
    d2 = jnp.maximum(sq_r + sq_c - 2.0 * dot, 0.0)
    dist = jnp.sqrt(d2)

    row_ids = i * block_rows + jax.lax.broadcasted_iota(
        jnp.int32, dist.shape, 0)
    col_ids = jax.lax.broadcasted_iota(jnp.int32, dist.shape, 1)
    dist = jnp.where(row_ids == col_ids, _BIG, dist)

    match = lab_row_ref[...] == lab_col_ref[...]

    t = jnp.min(dist, axis=1, keepdims=True)
    d0 = t
    tmp = dist
    for _ in range(_K - 1):
        tmp = jnp.where(tmp == t, _BIG, tmp)
        t = jnp.min(tmp, axis=1, keepdims=True)

    sel = dist <= t
    cnt = jnp.sum(sel.astype(jnp.float32), axis=1, keepdims=True)
    selm = sel & match
    sum_m = jnp.sum(selm.astype(jnp.float32), axis=1, keepdims=True)
    sum_md = jnp.sum(jnp.where(selm, dist, 0.0), axis=1, keepdims=True)
    sum_e = jnp.sum(jnp.where(sel, jnp.exp(d0 - dist), 0.0),
                    axis=1, keepdims=True)

    lse = jnp.log(sum_e) - d0
    fast_sum = jnp.sum(sum_md + sum_m * lse).reshape(1, 1)

    exact = jnp.all(cnt == float(_K))

    @pl.when(i == 0)
    def _init():
        out_ref[...] = jnp.zeros((1, 1), jnp.float32)

    @pl.when(exact)
    def _fast():
        out_ref[...] += fast_sum

    @pl.when(jnp.logical_not(exact))
    def _slow():
        mcol = 2 * col_ids + 1 - match.astype(jnp.int32)
        dd = dist
        s_md = jnp.zeros_like(sq_r)
        s_m = jnp.zeros_like(sq_r)
        s_e = jnp.zeros_like(sq_r)
        dk = jnp.min(dd, axis=1, keepdims=True)
        for k in range(_K):
            first = jnp.min(jnp.where(dd == dk, mcol, _BIGI),
                            axis=1, keepdims=True)
            mk = (1 - (first & 1)).astype(jnp.float32)
            s_md += mk * dk
            s_m += mk
            s_e += jnp.exp(d0 - dk)
            if k < _K - 1:
                dd = jnp.where(mcol == first, _BIG, dd)
                dk = jnp.min(dd, axis=1, keepdims=True)
        lse_s = jnp.log(s_e) - d0
        out_ref[...] += jnp.sum(s_md + s_m * lse_s).reshape(1, 1)

    @pl.when(i == num_blocks - 1)
    def _finish():
        out_ref[...] = out_ref[...] / n_total


def kernel(batch, labels):
    n, d = batch.shape
    block_rows = 256
    num_blocks = n // block_rows
    lab_row = labels.reshape(n, 1)
    lab_col = labels.reshape(1, n)

    sq = pl.pallas_call(
        _row_norms_kernel,
        grid=(num_blocks,),
        in_specs=[pl.BlockSpec((block_rows, d), lambda i: (i, 0))],
        out_specs=pl.BlockSpec((block_rows, 1), lambda i: (i, 0)),
        out_shape=jax.ShapeDtypeStruct((n, 1), jnp.float32),
    )(batch)
    sq_col = sq.reshape(1, n)

    body = functools.partial(_knn_loss_kernel, n_total=n,
                             block_rows=block_rows, num_blocks=num_blocks)
    out = pl.pallas_call(
        body,
        grid=(num_blocks,),
        in_specs=[
            pl.BlockSpec((block_rows, d), lambda i: (i, 0)),
            pl.BlockSpec((n, d), lambda i: (0, 0)),
            pl.BlockSpec((1, n), lambda i: (0, 0)),
            pl.BlockSpec((block_rows, 1), lambda i: (i, 0)),
            pl.BlockSpec((1, n), lambda i: (0, 0)),
        ],
        out_specs=pl.BlockSpec((1, 1), lambda i: (0, 0)),
        out_shape=jax.ShapeDtypeStruct((1, 1), jnp.float32),
    )(batch, batch, sq_col, lab_row, lab_col)
    return out[0, 0]


# speedup vs baseline: 1.2451x; 1.0221x over previous
import functools

import jax
import jax.numpy as jnp
from jax.experimental import pallas as pl

_K = 5
_BIG = 3.0e38
_BIGI = 1 << 30


def _row_norms_kernel(x_ref, out_ref):
    x = x_ref[...]
    out_ref[...] = jnp.sum(x * x, axis=1, keepdims=True)


def _knn_loss_kernel(x_row_ref, x_full_ref, sq_col_ref,
                     lab_row_ref, lab_col_ref, out_ref,
                     *, n_total, block_rows, num_blocks):
    i = pl.program_id(0)
    x = x_row_ref[...]
    xf = x_full_ref[...]

    sq_r = jnp.sum(x * x, axis=1, keepdims=True)
    sq_c = sq_col_ref[...]

    dot = jax.lax.dot_general(x, xf, (((1,), (1,)), ((), ())),
                              preferred_element_type=jnp.float32)
    d2 = jnp.maximum(sq_r + sq_c - 2.0 * dot, 0.0)
    dist = jnp.sqrt(d2)

    rid = i * block_rows + jax.lax.broadcasted_iota(
        jnp.int32, (block_rows, 1), 0)
    cid = jax.lax.broadcasted_iota(jnp.int32, (1, dist.shape[1]), 1)
    dist = jnp.where(rid == cid, _BIG, dist)

    match = lab_row_ref[...] == lab_col_ref[...]

    t = jnp.min(dist, axis=1, keepdims=True)
    d0 = t
    tmp = dist
    for _ in range(_K - 1):
        tmp = jnp.where(tmp == t, _BIG, tmp)
        t = jnp.min(tmp, axis=1, keepdims=True)

    sel = dist <= t
    cm = jnp.sum(jnp.where(sel, jnp.where(match, 8193.0, 1.0), 0.0),
                 axis=1, keepdims=True)
    sum_m = jnp.floor(cm * (1.0 / 8192.0))
    cnt = cm - 8192.0 * sum_m
    sum_md = jnp.sum(jnp.where(sel & match, dist, 0.0),
                     axis=1, keepdims=True)
    sum_e = jnp.sum(jnp.where(sel, jnp.exp(d0 - dist), 0.0),
                    axis=1, keepdims=True)

    lse = jnp.log(sum_e) - d0
    fast_sum = jnp.sum(sum_md + sum_m * lse).reshape(1, 1)

    exact = jnp.all(cnt == float(_K))

    @pl.when(i == 0)
    def _init():
        out_ref[...] = jnp.zeros((1, 1), jnp.float32)

    @pl.when(exact)
    def _fast():
        out_ref[...] += fast_sum

    @pl.when(jnp.logical_not(exact))
    def _slow():
        col_ids = jax.lax.broadcasted_iota(jnp.int32, dist.shape, 1)
        mcol = 2 * col_ids + 1 - match.astype(jnp.int32)
        dd = dist
        s_md = jnp.zeros_like(sq_r)
        s_m = jnp.zeros_like(sq_r)
        s_e = jnp.zeros_like(sq_r)
        dk = jnp.min(dd, axis=1, keepdims=True)
        for k in range(_K):
            first = jnp.min(jnp.where(dd == dk, mcol, _BIGI),
                            axis=1, keepdims=True)
            mk = (1 - (first & 1)).astype(jnp.float32)
            s_md += mk * dk
            s_m += mk
            s_e += jnp.exp(d0 - dk)
            if k < _K - 1:
                dd = jnp.where(mcol == first, _BIG, dd)
                dk = jnp.min(dd, axis=1, keepdims=True)
        lse_s = jnp.log(s_e) - d0
        out_ref[...] += jnp.sum(s_md + s_m * lse_s).reshape(1, 1)

    @pl.when(i == num_blocks - 1)
    def _finish():
        out_ref[...] = out_ref[...] / n_total


def kernel(batch, labels):
    n, d = batch.shape
    block_rows = 256
    num_blocks = n // block_rows
    lab_row = labels.reshape(n, 1)
    lab_col = labels.reshape(1, n)

    sq = pl.pallas_call(
        _row_norms_kernel,
        grid=(num_blocks,),
        in_specs=[pl.BlockSpec((block_rows, d), lambda i: (i, 0))],
        out_specs=pl.BlockSpec((block_rows, 1), lambda i: (i, 0)),
        out_shape=jax.ShapeDtypeStruct((n, 1), jnp.float32),
    )(batch)
    sq_col = sq.reshape(1, n)

    body = functools.partial(_knn_loss_kernel, n_total=n,
                             block_rows=block_rows, num_blocks=num_blocks)
    out = pl.pallas_call(
        body,
        grid=(num_blocks,),
        in_specs=[
            pl.BlockSpec((block_rows, d), lambda i: (i, 0)),
            pl.BlockSpec((n, d), lambda i: (0, 0)),
            pl.BlockSpec((1, n), lambda i: (0, 0)),
            pl.BlockSpec((block_rows, 1), lambda i: (i, 0)),
            pl.BlockSpec((1, n), lambda i: (0, 0)),
        ],
        out_specs=pl.BlockSpec((1, 1), lambda i: (0, 0)),
        out_shape=jax.ShapeDtypeStruct((1, 1), jnp.float32),
    )(batch, batch, sq_col, lab_row, lab_col)
    return out[0, 0]


# speedup vs baseline: 1.3486x; 1.0831x over previous
import functools

import jax
import jax.numpy as jnp
from jax.experimental import pallas as pl

_K = 5
_BIG = 3.0e38
_BIGI = 1 << 30


def _row_norms_kernel(x_ref, out_ref):
    x = x_ref[...]
    out_ref[...] = jnp.sum(x * x, axis=1, keepdims=True)


def _knn_loss_kernel(x_row_ref, x_full_ref, sq_col_ref,
                     lab_row_ref, lab_col_ref, out_ref,
                     *, n_total, block_rows, num_blocks):
    i = pl.program_id(0)
    x = x_row_ref[...]
    xf = x_full_ref[...]

    sq_r = jnp.sum(x * x, axis=1, keepdims=True)
    sq_c = sq_col_ref[...]

    dot = jax.lax.dot_general(x, xf, (((1,), (1,)), ((), ())),
                              preferred_element_type=jnp.float32)
    d2 = jnp.maximum(sq_r + sq_c - 2.0 * dot, 0.0)
    dist = jnp.sqrt(d2)

    rid = i * block_rows + jax.lax.broadcasted_iota(
        jnp.int32, (block_rows, 1), 0)
    cid = jax.lax.broadcasted_iota(jnp.int32, (1, dist.shape[1]), 1)
    dist = jnp.where(rid == cid, _BIG, dist)

    match = lab_row_ref[...] == lab_col_ref[...]

    t = jnp.min(dist, axis=1, keepdims=True)
    d0 = t
    tmp = dist
    for _ in range(_K - 1):
        tmp = jnp.where(tmp == t, _BIG, tmp)
        t = jnp.min(tmp, axis=1, keepdims=True)

    sel = dist <= t
    cm = jnp.sum(jnp.where(sel, jnp.where(match, 8193.0, 1.0), 0.0),
                 axis=1, keepdims=True)
    sum_m = jnp.floor(cm * (1.0 / 8192.0))
    cnt = cm - 8192.0 * sum_m
    sum_md = jnp.sum(jnp.where(sel & match, dist, 0.0),
                     axis=1, keepdims=True)
    sum_e = jnp.sum(jnp.where(sel, jnp.exp(d0 - dist), 0.0),
                    axis=1, keepdims=True)

    lse = jnp.log(sum_e) - d0
    fast_sum = jnp.sum(sum_md + sum_m * lse).reshape(1, 1)

    exact = jnp.all(cnt == float(_K))

    @pl.when(i == 0)
    def _init():
        out_ref[...] = jnp.zeros((1, 1), jnp.float32)

    @pl.when(exact)
    def _fast():
        out_ref[...] += fast_sum

    @pl.when(jnp.logical_not(exact))
    def _slow():
        col_ids = jax.lax.broadcasted_iota(jnp.int32, dist.shape, 1)
        mcol = 2 * col_ids + 1 - match.astype(jnp.int32)
        dd = dist
        s_md = jnp.zeros_like(sq_r)
        s_m = jnp.zeros_like(sq_r)
        s_e = jnp.zeros_like(sq_r)
        dk = jnp.min(dd, axis=1, keepdims=True)
        for k in range(_K):
            first = jnp.min(jnp.where(dd == dk, mcol, _BIGI),
                            axis=1, keepdims=True)
            mk = (1 - (first & 1)).astype(jnp.float32)
            s_md += mk * dk
            s_m += mk
            s_e += jnp.exp(d0 - dk)
            if k < _K - 1:
                dd = jnp.where(mcol == first, _BIG, dd)
                dk = jnp.min(dd, axis=1, keepdims=True)
        lse_s = jnp.log(s_e) - d0
        out_ref[...] += jnp.sum(s_md + s_m * lse_s).reshape(1, 1)

    @pl.when(i == num_blocks - 1)
    def _finish():
        out_ref[...] = out_ref[...] / n_total


def kernel(batch, labels):
    n, d = batch.shape
    block_rows = 512
    num_blocks = n // block_rows
    lab_row = labels.reshape(n, 1)
    lab_col = labels.reshape(1, n)

    sq = pl.pallas_call(
        _row_norms_kernel,
        grid=(num_blocks,),
        in_specs=[pl.BlockSpec((block_rows, d), lambda i: (i, 0))],
        out_specs=pl.BlockSpec((block_rows, 1), lambda i: (i, 0)),
        out_shape=jax.ShapeDtypeStruct((n, 1), jnp.float32),
    )(batch)
    sq_col = sq.reshape(1, n)

    body = functools.partial(_knn_loss_kernel, n_total=n,
                             block_rows=block_rows, num_blocks=num_blocks)
    out = pl.pallas_call(
        body,
        grid=(num_blocks,),
        in_specs=[
            pl.BlockSpec((block_rows, d), lambda i: (i, 0)),
            pl.BlockSpec((n, d), lambda i: (0, 0)),
            pl.BlockSpec((1, n), lambda i: (0, 0)),
            pl.BlockSpec((block_rows, 1), lambda i: (i, 0)),
            pl.BlockSpec((1, n), lambda i: (0, 0)),
        ],
        out_specs=pl.BlockSpec((1, 1), lambda i: (0, 0)),
        out_shape=jax.ShapeDtypeStruct((1, 1), jnp.float32),
    )(batch, batch, sq_col, lab_row, lab_col)
    return out[0, 0]
